# Initial kernel scaffold; baseline (speedup 1.0000x reference)
import jax
import jax.numpy as jnp
from jax.experimental import pallas as pl


def kernel(x, target):
    raise NotImplementedError("write your pallas kernel here")


# speedup vs baseline: 5.9359x; 5.9359x over previous
import functools
import math

import jax
import jax.numpy as jnp
import numpy as np
from jax.experimental import pallas as pl
from jax.experimental.pallas import tpu as pltpu

_SIZE = 100000
_PAD = 0
_SMOOTH = 0.1
_CONF = 1.0 - _SMOOTH
_EPS = float(np.float32(_SMOOTH / (_SIZE - 2)))
_C = _CONF * math.log(_CONF) + (_SIZE - 2) * _EPS * math.log(_EPS)


def _ls_kernel(t_ref, x_ref, o_ref, acc_ref):
    b = pl.program_id(0)
    nb = pl.num_programs(0)

    @pl.when(b == 0)
    def _init():
        acc_ref[0] = 0.0
        acc_ref[1] = 0.0
        acc_ref[2] = 0.0

    x = x_ref[0]
    t = t_ref[0]
    cols = jax.lax.broadcasted_iota(jnp.int32, x.shape, 1)
    w = jnp.where(cols == t, np.float32(_CONF), np.float32(_EPS))
    w = jnp.where(cols == 0, np.float32(0.0), w)
    w = jnp.where(t != 0, w, np.float32(0.0))
    acc_ref[0] += jnp.sum(w * x)

    m = (t != 0).astype(jnp.float32)
    rows = jax.lax.broadcasted_iota(jnp.int32, t.shape, 0)
    acc_ref[1] += jnp.sum(m)
    acc_ref[2] += jnp.sum(jnp.where(rows != 0, m, 0.0))

    @pl.when(b == nb - 1)
    def _fin():
        kl = acc_ref[1] * np.float32(_C) - acc_ref[0]
        o_ref[0, 0] = kl / acc_ref[2]


@jax.jit
def _label_smoothing_loss(x, target):
    B, S, V = x.shape
    t3 = target.reshape(B, S, 1)
    out = pl.pallas_call(
        _ls_kernel,
        grid=(B,),
        in_specs=[
            pl.BlockSpec((1, S, 1), lambda b: (b, 0, 0)),
            pl.BlockSpec((1, S, V), lambda b: (b, 0, 0)),
        ],
        out_specs=pl.BlockSpec(memory_space=pltpu.SMEM),
        out_shape=jax.ShapeDtypeStruct((1, 1), jnp.float32),
        scratch_shapes=[pltpu.SMEM((3,), jnp.float32)],
        compiler_params=pltpu.CompilerParams(
            dimension_semantics=("arbitrary",),
        ),
    )(t3, x)
    return out.reshape(())


def kernel(x, target):
    return _label_smoothing_loss(x, target)
